# baseline (speedup 1.0000x reference)
import jax
import jax.numpy as jnp
from jax.experimental import pallas as pl
from jax.experimental.pallas import tpu as pltpu

_NUM_EDGES = 800000
_EMB_DIM = 64


def _noop(out_ref):
    pass


def kernel(material_id, num_edges, table):
    del num_edges, material_id, table
    out = pl.pallas_call(
        _noop,
        out_specs=pl.BlockSpec(memory_space=pl.ANY),
        out_shape=jax.ShapeDtypeStruct((_NUM_EDGES // 2, 128), jnp.float32),
    )()
    return jnp.reshape(out, (_NUM_EDGES, _EMB_DIM))


# speedup vs baseline: 1.0446x; 1.0446x over previous
import functools

import jax
import jax.numpy as jnp
from jax import lax
from jax.experimental import pallas as pl
from jax.experimental.pallas import tpu as pltpu
from jax.experimental.pallas import tpu_sc as plsc

_NUM_EDGES = 800000
_EMB_DIM = 64
_NC = 2
_NS = 16
_NW = _NC * _NS
_ROWS_PER_W = _NUM_EDGES // _NW
_BUF_ROWS = 1000
_N_COPIES = _ROWS_PER_W // _BUF_ROWS
_FIRE = 13


def _tc_lookup(mid_ref, table_ref, out_ref):
    r = mid_ref[0] % 8
    row = table_ref[pl.ds(r, 1), :]
    row2 = jnp.concatenate([row, row], axis=1)
    out_ref[...] = jnp.broadcast_to(row2, out_ref.shape)


def _sc_body(row8_hbm, out_hbm, row_v, buf_v, sem_o):
    wid = lax.axis_index("s") * _NC + lax.axis_index("c")
    base = wid * _ROWS_PER_W

    pltpu.sync_copy(row8_hbm, row_v)
    r0 = row_v[0, pl.ds(0, 16)]
    r1 = row_v[0, pl.ds(16, 16)]
    r2 = row_v[0, pl.ds(32, 16)]
    r3 = row_v[0, pl.ds(48, 16)]

    def fill(i, carry):
        a, b, c, d = carry
        buf_v[i, pl.ds(0, 16)] = a
        buf_v[i, pl.ds(16, 16)] = b
        buf_v[i, pl.ds(32, 16)] = c
        buf_v[i, pl.ds(48, 16)] = d
        return carry

    lax.fori_loop(0, _BUF_ROWS, fill, (r0, r1, r2, r3), unroll=2)

    for g in range(0, _N_COPIES, _FIRE):
        copies = [
            pltpu.make_async_copy(
                buf_v,
                out_hbm.at[pl.ds(base + k * _BUF_ROWS, _BUF_ROWS), :],
                sem_o,
            )
            for k in range(g, min(g + _FIRE, _N_COPIES))
        ]
        for c in copies:
            c.start()
        for c in copies:
            c.wait()


def kernel(material_id, num_edges, table):
    del num_edges
    row8 = pl.pallas_call(
        _tc_lookup,
        grid_spec=pltpu.PrefetchScalarGridSpec(
            num_scalar_prefetch=1,
            grid=(1,),
            in_specs=[
                pl.BlockSpec((8, _EMB_DIM), lambda i, mid: (mid[0] // 8, 0)),
            ],
            out_specs=pl.BlockSpec((8, 128), lambda i, mid: (0, 0)),
        ),
        out_shape=jax.ShapeDtypeStruct((8, 128), jnp.float32),
    )(material_id, table)

    mesh = plsc.VectorSubcoreMesh(core_axis_name="c", subcore_axis_name="s")
    kern = functools.partial(
        pl.kernel,
        mesh=mesh,
        out_type=jax.ShapeDtypeStruct((_NUM_EDGES, _EMB_DIM), jnp.float32),
        scratch_types=[
            pltpu.VMEM((8, 128), jnp.float32),
            pltpu.VMEM((_BUF_ROWS, _EMB_DIM), jnp.float32),
            pltpu.SemaphoreType.DMA,
        ],
    )(_sc_body)
    return kern(row8)


# speedup vs baseline: 1.3392x; 1.2820x over previous
import functools

import jax
import jax.numpy as jnp
from jax import lax
from jax.experimental import pallas as pl
from jax.experimental.pallas import tpu as pltpu
from jax.experimental.pallas import tpu_sc as plsc

_NUM_EDGES = 800000
_EMB_DIM = 64
_NC = 2
_NS = 16
_NW = _NC * _NS
_ROWS_PER_W = _NUM_EDGES // _NW
_BUF_ROWS = 1000
_N_COPIES = _ROWS_PER_W // _BUF_ROWS
_FIRE = 13


def _tc_lookup(mid_ref, table_ref, out_ref):
    r = mid_ref[0] % 8
    row = table_ref[pl.ds(r, 1), :]
    row2 = jnp.concatenate([row, row], axis=1)
    out_ref[...] = jnp.broadcast_to(row2, out_ref.shape)


def _sc_body(row8_hbm, out_hbm, row_v, buf_v, sem_o):
    wid = lax.axis_index("s") * _NC + lax.axis_index("c")
    base = wid * _ROWS_PER_W

    pltpu.sync_copy(row8_hbm, row_v)
    regs = tuple(row_v[0, pl.ds(16 * j, 16)] for j in range(8))

    def fill(i, carry):
        for j in range(8):
            buf_v[i, pl.ds(16 * j, 16)] = carry[j]
        return carry

    lax.fori_loop(0, _BUF_ROWS, fill, regs, unroll=2)

    for g in range(0, _N_COPIES, _FIRE):
        copies = [
            pltpu.make_async_copy(
                buf_v,
                out_hbm.at[pl.ds(base + k * _BUF_ROWS, _BUF_ROWS), :],
                sem_o,
            )
            for k in range(g, min(g + _FIRE, _N_COPIES))
        ]
        for c in copies:
            c.start()
        for c in copies:
            c.wait()


def kernel(material_id, num_edges, table):
    del num_edges
    row8 = pl.pallas_call(
        _tc_lookup,
        grid_spec=pltpu.PrefetchScalarGridSpec(
            num_scalar_prefetch=1,
            grid=(1,),
            in_specs=[
                pl.BlockSpec((8, _EMB_DIM), lambda i, mid: (mid[0] // 8, 0)),
            ],
            out_specs=pl.BlockSpec((8, 128), lambda i, mid: (0, 0)),
        ),
        out_shape=jax.ShapeDtypeStruct((8, 128), jnp.float32),
    )(material_id, table)

    mesh = plsc.VectorSubcoreMesh(core_axis_name="c", subcore_axis_name="s")
    kern = functools.partial(
        pl.kernel,
        mesh=mesh,
        out_type=jax.ShapeDtypeStruct((_NUM_EDGES, 128), jnp.float32),
        scratch_types=[
            pltpu.VMEM((8, 128), jnp.float32),
            pltpu.VMEM((_BUF_ROWS, 128), jnp.float32),
            pltpu.SemaphoreType.DMA,
        ],
    )(_sc_body)
    wide = kern(row8)
    return lax.slice(wide, (0, 0), (_NUM_EDGES, _EMB_DIM))


# speedup vs baseline: 1.3457x; 1.0049x over previous
import functools

import jax
import jax.numpy as jnp
from jax import lax
from jax.experimental import pallas as pl
from jax.experimental.pallas import tpu as pltpu
from jax.experimental.pallas import tpu_sc as plsc

_NUM_EDGES = 800000
_EMB_DIM = 64
_NC = 2
_NS = 16
_NW = _NC * _NS
_ROWS_PER_W = _NUM_EDGES // _NW
_BUF_ROWS = 1000
_N_COPIES = _ROWS_PER_W // _BUF_ROWS
_FIRE = 25


def _tc_lookup(mid_ref, table_ref, out_ref):
    r = mid_ref[0] % 8
    row = table_ref[pl.ds(r, 1), :]
    row2 = jnp.concatenate([row, row], axis=1)
    out_ref[...] = jnp.broadcast_to(row2, out_ref.shape)


def _sc_body(row8_hbm, out_hbm, row_v, buf_v, sem_o):
    wid = lax.axis_index("s") * _NC + lax.axis_index("c")
    base = wid * _ROWS_PER_W

    pltpu.sync_copy(row8_hbm, row_v)
    regs = tuple(row_v[0, pl.ds(16 * j, 16)] for j in range(8))

    def fill(i, carry):
        for j in range(8):
            buf_v[i, pl.ds(16 * j, 16)] = carry[j]
        return carry

    lax.fori_loop(0, _BUF_ROWS, fill, regs, unroll=4)

    for g in range(0, _N_COPIES, _FIRE):
        copies = [
            pltpu.make_async_copy(
                buf_v,
                out_hbm.at[pl.ds(base + k * _BUF_ROWS, _BUF_ROWS), :],
                sem_o,
            )
            for k in range(g, min(g + _FIRE, _N_COPIES))
        ]
        for c in copies:
            c.start()
        for c in copies:
            c.wait()


def kernel(material_id, num_edges, table):
    del num_edges
    row8 = pl.pallas_call(
        _tc_lookup,
        grid_spec=pltpu.PrefetchScalarGridSpec(
            num_scalar_prefetch=1,
            grid=(1,),
            in_specs=[
                pl.BlockSpec((8, _EMB_DIM), lambda i, mid: (mid[0] // 8, 0)),
            ],
            out_specs=pl.BlockSpec((8, 128), lambda i, mid: (0, 0)),
        ),
        out_shape=jax.ShapeDtypeStruct((8, 128), jnp.float32),
    )(material_id, table)

    mesh = plsc.VectorSubcoreMesh(core_axis_name="c", subcore_axis_name="s")
    kern = functools.partial(
        pl.kernel,
        mesh=mesh,
        out_type=jax.ShapeDtypeStruct((_NUM_EDGES, 128), jnp.float32),
        scratch_types=[
            pltpu.VMEM((8, 128), jnp.float32),
            pltpu.VMEM((_BUF_ROWS, 128), jnp.float32),
            pltpu.SemaphoreType.DMA,
        ],
    )(_sc_body)
    wide = kern(row8)
    return lax.slice(wide, (0, 0), (_NUM_EDGES, _EMB_DIM))


# speedup vs baseline: 5.0991x; 3.7891x over previous
import functools

import jax
import jax.numpy as jnp
from jax import lax
from jax.experimental import pallas as pl
from jax.experimental.pallas import tpu as pltpu
from jax.experimental.pallas import tpu_sc as plsc

_NUM_EDGES = 800000
_EMB_DIM = 64
_NW = 32
_BUF_COLS = 1280
_N_COL_CHUNKS = _NUM_EDGES // _BUF_COLS
_N_ROW_BLOCKS = _EMB_DIM // 8
_N_CHUNKS = _N_ROW_BLOCKS * _N_COL_CHUNKS
_PER_TILE = -(-_N_CHUNKS // _NW)
_FIRE = 16


def _tc_lookup(mid_ref, tableT_ref, out_ref):
    c = mid_ref[0] % 128
    lane = jax.lax.broadcasted_iota(jnp.int32, (1, 128), 1)
    masked = jnp.where(lane == c, tableT_ref[...], 0.0)
    col = jnp.sum(masked, axis=1, keepdims=True)
    out_ref[...] = jnp.broadcast_to(col, out_ref.shape)


def _sc_body(stage_hbm, out_hbm, stage_v, buf_v, sem_o):
    wid = lax.axis_index("s") * 2 + lax.axis_index("c")

    pltpu.sync_copy(stage_hbm, stage_v)
    regs = tuple(stage_v[j, pl.ds(0, 16)] for j in range(_EMB_DIM))

    def fill(t, carry):
        for j in range(_EMB_DIM):
            buf_v[j, pl.ds(t * 16, 16)] = carry[j]
        return carry

    lax.fori_loop(0, _BUF_COLS // 16, fill, regs)

    first = wid * _PER_TILE
    for g in range(0, _PER_TILE, _FIRE):
        copies = []
        for t in range(g, min(g + _FIRE, _PER_TILE)):
            cp = jnp.minimum(first + t, _N_CHUNKS - 1)
            rb = pl.multiple_of((cp // _N_COL_CHUNKS) * 8, 8)
            cc = pl.multiple_of((cp % _N_COL_CHUNKS) * _BUF_COLS, _BUF_COLS)
            copies.append(
                pltpu.make_async_copy(
                    buf_v.at[pl.ds(rb, 8), :],
                    out_hbm.at[pl.ds(rb, 8), pl.ds(cc, _BUF_COLS)],
                    sem_o,
                )
            )
        for cpy in copies:
            cpy.start()
        for cpy in copies:
            cpy.wait()


def kernel(material_id, num_edges, table):
    del num_edges
    tableT = jnp.swapaxes(table, 0, 1)

    stage = pl.pallas_call(
        _tc_lookup,
        grid_spec=pltpu.PrefetchScalarGridSpec(
            num_scalar_prefetch=1,
            grid=(1,),
            in_specs=[
                pl.BlockSpec((_EMB_DIM, 128), lambda i, mid: (0, mid[0] // 128)),
            ],
            out_specs=pl.BlockSpec((_EMB_DIM, 16), lambda i, mid: (0, 0)),
        ),
        out_shape=jax.ShapeDtypeStruct((_EMB_DIM, 16), jnp.float32),
    )(material_id, tableT)

    mesh = plsc.VectorSubcoreMesh(core_axis_name="c", subcore_axis_name="s")
    kern = functools.partial(
        pl.kernel,
        mesh=mesh,
        out_type=jax.ShapeDtypeStruct((_EMB_DIM, _NUM_EDGES), jnp.float32),
        scratch_types=[
            pltpu.VMEM((_EMB_DIM, 16), jnp.float32),
            pltpu.VMEM((_EMB_DIM, _BUF_COLS), jnp.float32),
            pltpu.SemaphoreType.DMA,
        ],
    )(_sc_body)
    wide = kern(stage)
    return jnp.swapaxes(wide, 0, 1)


# speedup vs baseline: 5.5336x; 1.0852x over previous
import functools

import jax
import jax.numpy as jnp
from jax import lax
from jax.experimental import pallas as pl
from jax.experimental.pallas import tpu as pltpu
from jax.experimental.pallas import tpu_sc as plsc

_NUM_EDGES = 800000
_EMB_DIM = 64
_NW = 32
_BUF_COLS = 6400
_N_COL_CHUNKS = _NUM_EDGES // _BUF_COLS
_TILES_PER_RB = 4
_PER_TILE = 32
_FIRE = 16


def _tc_lookup(mid_ref, tableT_ref, out_ref):
    c = mid_ref[0] % 128
    lane = jax.lax.broadcasted_iota(jnp.int32, (1, 128), 1)
    masked = jnp.where(lane == c, tableT_ref[...], 0.0)
    col = jnp.sum(masked, axis=1, keepdims=True)
    out_ref[...] = jnp.broadcast_to(col, out_ref.shape)


def _sc_body(stage_hbm, out_hbm, stage_v, buf_v, sem_o):
    wid = lax.axis_index("s") * 2 + lax.axis_index("c")
    rb = pl.multiple_of((wid % 8) * 8, 8)
    q = wid // 8

    pltpu.sync_copy(stage_hbm, stage_v)
    regs = tuple(stage_v[rb + j, pl.ds(0, 16)] for j in range(8))

    def fill(t, carry):
        for j in range(8):
            buf_v[j, pl.ds(t * 16, 16)] = carry[j]
        return carry

    lax.fori_loop(0, _BUF_COLS // 16, fill, regs)

    first = q * 31
    for g in range(0, _PER_TILE, _FIRE):
        copies = []
        for t in range(g, min(g + _FIRE, _PER_TILE)):
            cc = pl.multiple_of(
                jnp.minimum(first + t, _N_COL_CHUNKS - 1) * _BUF_COLS, _BUF_COLS
            )
            copies.append(
                pltpu.make_async_copy(
                    buf_v,
                    out_hbm.at[pl.ds(rb, 8), pl.ds(cc, _BUF_COLS)],
                    sem_o,
                )
            )
        for cpy in copies:
            cpy.start()
        for cpy in copies:
            cpy.wait()


def kernel(material_id, num_edges, table):
    del num_edges
    tableT = jnp.swapaxes(table, 0, 1)

    stage = pl.pallas_call(
        _tc_lookup,
        grid_spec=pltpu.PrefetchScalarGridSpec(
            num_scalar_prefetch=1,
            grid=(1,),
            in_specs=[
                pl.BlockSpec((_EMB_DIM, 128), lambda i, mid: (0, mid[0] // 128)),
            ],
            out_specs=pl.BlockSpec((_EMB_DIM, 16), lambda i, mid: (0, 0)),
        ),
        out_shape=jax.ShapeDtypeStruct((_EMB_DIM, 16), jnp.float32),
    )(material_id, tableT)

    mesh = plsc.VectorSubcoreMesh(core_axis_name="c", subcore_axis_name="s")
    kern = functools.partial(
        pl.kernel,
        mesh=mesh,
        out_type=jax.ShapeDtypeStruct((_EMB_DIM, _NUM_EDGES), jnp.float32),
        scratch_types=[
            pltpu.VMEM((_EMB_DIM, 16), jnp.float32),
            pltpu.VMEM((8, _BUF_COLS), jnp.float32),
            pltpu.SemaphoreType.DMA,
        ],
    )(_sc_body)
    wide = kern(stage)
    return jnp.swapaxes(wide, 0, 1)
